# baseline (speedup 1.0000x reference)
import functools

import jax
import jax.numpy as jnp
import numpy as np
from jax._src.pallas.mosaic import core as _tpu_core
from jax import lax
from jax.experimental import pallas as pl
from jax.experimental.pallas import tpu as pltpu
from jax.experimental.pallas import tpu_sc as plsc

_E = 256
_C = 128
_S = 64
_M = _E // _S
_ROW = _M * _C
_NREP = 64
_BBLK = 64


def _body(sel_ref, lat_ref, embt_ref, out_ref):
    sel = sel_ref[...]
    jcol = jax.lax.broadcasted_iota(jnp.int32, (_BBLK, _NREP), 1)
    for s in range(_S):
        onehot = (sel[:, s][:, None] == jcol).astype(jnp.bfloat16)
        g = jax.lax.dot_general(
            onehot, embt_ref[s],
            (((1,), (0,)), ((), ())),
            preferred_element_type=jnp.float32,
        )
        sl = slice(_M * s, _M * (s + 1))
        out_ref[:, sl, :] = lat_ref[:, sl, :] + g.reshape(_BBLK, _M, _C)


_NW = 32
_TAB_ROWS = _E * _NREP


_QROWS = 64
_NBUF = 4


def _sc_body(lat_hbm, sel_hbm, tab_hbm, out_hbm,
             sel_v, idx_v, lat_v, emb_v, sem_l, sem_g, sem_o,
             b0=0, nb=None, ob0=0):
    wid = lax.axis_index("s") * 2 + lax.axis_index("c")
    b_per_w = (nb if nb is not None else lat_hbm.shape[0]) // _NW
    base = b0 + wid * b_per_w
    nsteps = b_per_w

    pltpu.sync_copy(sel_hbm.at[pl.ds(base, b_per_w)], sel_v)

    lane = lax.iota(jnp.int32, 16)
    rep2 = lax.shift_right_logical(lane, 1)
    gdn = lax.GatherDimensionNumbers(
        offset_dims=(), collapsed_slice_dims=(0,), start_index_map=(0,))

    def start(t, q):
        b = base + t
        s16 = sel_v[t, pl.ds(q * 16, 16)]
        for c in range(2):
            sval = lax.gather(s16, (8 * c + rep2)[:, None], gdn, (1,),
                              mode=lax.GatherScatterMode.PROMISE_IN_BOUNDS)
            ip = (q * (_QROWS // 2) + c * 16) + lane
            idx_v[q, pl.ds(c * 16, 16)] = ip * _NREP + sval
        pltpu.async_copy(
            lat_hbm.at[b, pl.ds(q * _QROWS, _QROWS)], lat_v.at[q], sem_l.at[q])
        pltpu.async_copy(tab_hbm.at[idx_v.at[q]], emb_v.at[q], sem_g.at[q])

    def finish(t, q):
        b = base + t
        pltpu.make_async_copy(
            lat_hbm.at[b, pl.ds(q * _QROWS, _QROWS)], lat_v.at[q],
            sem_l.at[q]).wait()
        pltpu.make_async_copy(
            tab_hbm.at[idx_v.at[q]], emb_v.at[q], sem_g.at[q]).wait()

        def addrow(p, inner):
            for half in range(2):
                row = 2 * p + half
                for ch in range(4):
                    ew = emb_v[q, p, pl.ds(half * 64 + ch * 16, 16)]
                    eb = plsc.bitcast(ew, jnp.bfloat16)
                    ea, ebb = plsc.unpack(
                        eb, format=plsc.PackFormat.INTERLEAVED,
                        preferred_element_type=jnp.float32)
                    sl0 = pl.ds(ch * 32, 16)
                    sl1 = pl.ds(ch * 32 + 16, 16)
                    lat_v[q, row, sl0] = lat_v[q, row, sl0] + ea
                    lat_v[q, row, sl1] = lat_v[q, row, sl1] + ebb
            return inner

        lax.fori_loop(0, _QROWS // 2, addrow, 0)
        pltpu.async_copy(
            lat_v.at[q],
            out_hbm.at[b - ob0, pl.ds(q * _QROWS, _QROWS)], sem_o.at[q])

    def drain_out(t, q):
        pltpu.make_async_copy(
            lat_v.at[q], out_hbm.at[base - ob0 + t, pl.ds(q * _QROWS, _QROWS)],
            sem_o.at[q]).wait()

    start(0, 0)
    start(0, 1)
    start(0, 2)

    def step(t, carry):
        for p in range(4):
            finish(t, p)
            if p == 0:
                @pl.when(t >= 1)
                def _():
                    drain_out(t - 1, 3)
            else:
                drain_out(t, p - 1)
            if p == 0:
                start(t, 3)
            else:
                @pl.when(t < nsteps - 1)
                def _():
                    start(t + 1, p - 1)
        return carry

    lax.fori_loop(0, nsteps, step, 0)
    drain_out(nsteps - 1, 3)


def _sc_call(latents, sel, emb, b0=0, nb=None):
    if nb is None:
        nb = latents.shape[0]
    ile = (
        emb.astype(jnp.bfloat16)
        .reshape(_E, _NREP, _C // 32, 2, 16)
        .transpose(0, 1, 2, 4, 3)
        .reshape(_E, _NREP, _C)
    )
    pairs = (
        ile.reshape(_E // 2, 2, _NREP, _C)
        .transpose(0, 2, 1, 3)
        .reshape(_E // 2 * _NREP, 2 * _C)
    )
    tab = jax.lax.bitcast_convert_type(
        pairs.reshape(_E // 2 * _NREP, _C, 2), jnp.int32)
    b_per_w = nb // _NW
    body = functools.partial(_sc_body, b0=b0, nb=nb, ob0=b0)
    run = functools.partial(
        pl.kernel,
        mesh=plsc.VectorSubcoreMesh(core_axis_name="c", subcore_axis_name="s"),
        out_type=jax.ShapeDtypeStruct((nb, _E, _C), jnp.float32),
        scratch_types=[
            pltpu.VMEM((b_per_w, _S), jnp.int32),
            pltpu.VMEM((_NBUF, _QROWS // 2), jnp.int32),
            pltpu.VMEM((_NBUF, _QROWS, _C), jnp.float32),
            pltpu.VMEM((_NBUF, _QROWS // 2, _C), jnp.int32),
            pltpu.SemaphoreType.DMA((_NBUF,)),
            pltpu.SemaphoreType.DMA((_NBUF,)),
            pltpu.SemaphoreType.DMA((_NBUF,)),
        ],
        compiler_params=pltpu.CompilerParams(needs_layout_passes=False),
    )(body)
    return run(latents, sel, tab)


def kernel(latents, selections, emb):
    sel = selections.astype(jnp.int32)
    return _sc_call(latents, sel, emb)


def _tc_kernel_part(latents, selections, emb, nt):
    sel = selections.astype(jnp.int32)
    embt = (
        emb.reshape(_S, _M, _NREP, _C)
        .transpose(0, 2, 1, 3)
        .reshape(_S, _NREP, _ROW)
        .astype(jnp.bfloat16)
    )
    return pl.pallas_call(
        _body,
        grid=(nt // _BBLK,),
        in_specs=[
            pl.BlockSpec((_BBLK, _S), lambda i: (i, 0)),
            pl.BlockSpec((_BBLK, _E, _C), lambda i: (i, 0, 0)),
            pl.BlockSpec((_S, _NREP, _ROW), lambda i: (0, 0, 0)),
        ],
        out_specs=pl.BlockSpec((_BBLK, _E, _C), lambda i: (i, 0, 0)),
        out_shape=jax.ShapeDtypeStruct((nt, _E, _C), jnp.float32),
        compiler_params=pltpu.CompilerParams(
            dimension_semantics=("arbitrary",),
        ),
    )(sel, latents, embt)


def _tc_kernel(latents, selections, emb):
    b = latents.shape[0]
    sel = selections.astype(jnp.int32)
    embt = (
        emb.reshape(_S, _M, _NREP, _C)
        .transpose(0, 2, 1, 3)
        .reshape(_S, _NREP, _ROW)
        .astype(jnp.bfloat16)
    )
    return pl.pallas_call(
        _body,
        grid=(b // _BBLK,),
        in_specs=[
            pl.BlockSpec((_BBLK, _S), lambda i: (i, 0)),
            pl.BlockSpec((_BBLK, _E, _C), lambda i: (i, 0, 0)),
            pl.BlockSpec((_S, _NREP, _ROW), lambda i: (0, 0, 0)),
        ],
        out_specs=pl.BlockSpec((_BBLK, _E, _C), lambda i: (i, 0, 0)),
        out_shape=jax.ShapeDtypeStruct((b, _E, _C), jnp.float32),
        compiler_params=pltpu.CompilerParams(
            dimension_semantics=("arbitrary",),
        ),
    )(sel, latents, embt)


_NT = 512


class _HbmTensorCoreMesh(_tpu_core.TensorCoreMesh):
    @property
    def default_memory_space(self):
        return _tpu_core.MemorySpace.HBM


def _hy_tc_body(lat_hbm, sel_hbm, embt_hbm, tab_hbm, out_hbm,
                embt_v, selv, lat_v, out_v, sem_l, sem_o, sem_t,
                *sc_scratch):
    nblk = _NT // _BBLK
    pltpu.async_copy(embt_hbm, embt_v, sem_t).wait()
    pltpu.async_copy(sel_hbm.at[pl.ds(0, _NT)], selv, sem_t).wait()

    def lat_copy(t, par):
        return pltpu.make_async_copy(
            lat_hbm.at[pl.ds(t * _BBLK, _BBLK)], lat_v.at[par], sem_l.at[par])

    def out_copy(t, par):
        return pltpu.make_async_copy(
            out_v.at[par], out_hbm.at[pl.ds(t * _BBLK, _BBLK)], sem_o.at[par])

    lat_copy(0, 0).start()
    lat_copy(1, 1).start()
    jcol = jax.lax.broadcasted_iota(jnp.int32, (_BBLK, _NREP), 1)

    for t in range(nblk):
        par = t % 2
        lat_copy(t, par).wait()
        if t >= 2:
            out_copy(t - 2, par).wait()
        sel_blk = selv[pl.ds(t * _BBLK, _BBLK), :]
        for s in range(_S):
            onehot = (sel_blk[:, s][:, None] == jcol).astype(jnp.bfloat16)
            g = jax.lax.dot_general(
                onehot, embt_v[s], (((1,), (0,)), ((), ())),
                preferred_element_type=jnp.float32)
            sl = slice(_M * s, _M * (s + 1))
            out_v[par, :, sl, :] = lat_v[par, :, sl, :] + g.reshape(_BBLK, _M, _C)
        out_copy(t, par).start()
        if t + 2 < nblk:
            lat_copy(t + 2, par).start()
    out_copy(nblk - 2, (nblk - 2) % 2).wait()
    out_copy(nblk - 1, (nblk - 1) % 2).wait()


def _hy_sc_body(lat_hbm, sel_hbm, embt_hbm, tab_hbm, out_hbm,
                embt_v, selv, lat_v_tc, out_v_tc, sem_l_tc, sem_o_tc, sem_t,
                sel_v, idx_v, lat_v, emb_v, sem_l, sem_g, sem_o):
    _sc_body(lat_hbm, sel_hbm, tab_hbm, out_hbm,
             sel_v, idx_v, lat_v, emb_v, sem_l, sem_g, sem_o,
             b0=_NT, nb=lat_hbm.shape[0] - _NT, ob0=0)


def _hybrid_call(latents, sel, emb):
    nb = latents.shape[0]
    tab = emb.reshape(_TAB_ROWS, _C)
    embt = (
        emb.reshape(_S, _M, _NREP, _C)
        .transpose(0, 2, 1, 3)
        .reshape(_S, _NREP, _ROW)
        .astype(jnp.bfloat16)
    )
    b_per_w = (nb - _NT) // _NW
    tc_mesh = _HbmTensorCoreMesh(
        np.array([_tpu_core.TensorCore(0)], dtype=object), ("tc",))
    sc_mesh = plsc.VectorSubcoreMesh(core_axis_name="c", subcore_axis_name="s")
    vm_tc = pltpu.MemorySpace.VMEM @ tc_mesh
    sm_tc = pltpu.MemorySpace.SEMAPHORE @ tc_mesh
    vm_sc = pltpu.MemorySpace.VMEM @ sc_mesh
    sm_sc = pltpu.MemorySpace.SEMAPHORE @ sc_mesh
    dma = pltpu.SemaphoreType.DMA.dtype
    run = pl.kernel(
        body=[_hy_sc_body, _hy_tc_body],
        mesh=[sc_mesh, tc_mesh],
        out_type=jax.ShapeDtypeStruct((nb, _E, _C), jnp.float32),
        scratch_types=[
            vm_tc((_S, _NREP, _ROW), jnp.bfloat16),
            vm_tc((_NT, _S), jnp.int32),
            vm_tc((2, _BBLK, _E, _C), jnp.float32),
            vm_tc((2, _BBLK, _E, _C), jnp.float32),
            sm_tc((2,), dma),
            sm_tc((2,), dma),
            sm_tc((), dma),
            vm_sc((b_per_w, _S), jnp.int32),
            vm_sc((_NBUF, _QROWS), jnp.int32),
            vm_sc((_NBUF, _QROWS, _C), jnp.float32),
            vm_sc((_NBUF, _QROWS, _C), jnp.float32),
            sm_sc((_NBUF,), dma),
            sm_sc((_NBUF,), dma),
            sm_sc((_NBUF,), dma),
        ],
    )
    return run(latents, sel, embt, tab)


# speedup vs baseline: 1.7580x; 1.7580x over previous
import functools

import jax
import jax.numpy as jnp
from jax import lax
from jax.experimental import pallas as pl
from jax.experimental.pallas import tpu as pltpu
from jax.experimental.pallas import tpu_sc as plsc

_E = 256
_C = 128
_S = 64
_M = _E // _S
_ROW = _M * _C
_NREP = 64
_BBLK = 64


def _body(sel_ref, lat_ref, embt_ref, out_ref):
    sel = sel_ref[...]
    jcol = jax.lax.broadcasted_iota(jnp.int32, (_BBLK, _NREP), 1)
    for s in range(_S):
        onehot = (sel[:, s][:, None] == jcol).astype(jnp.bfloat16)
        g = jax.lax.dot_general(
            onehot, embt_ref[s],
            (((1,), (0,)), ((), ())),
            preferred_element_type=jnp.float32,
        )
        sl = slice(_M * s, _M * (s + 1))
        out_ref[:, sl, :] = lat_ref[:, sl, :] + g.reshape(_BBLK, _M, _C)


_NW = 32
_TAB_ROWS = _E * _NREP


_QROWS = 64
_NBUF = 4


def _sc_body(lat_hbm, sel_hbm, tab_hbm, out_hbm,
             sel_v, idx_v, lat_v, emb_v, sem_l, sem_g, sem_o):
    wid = lax.axis_index("s") * 2 + lax.axis_index("c")
    b_per_w = lat_hbm.shape[0] // _NW
    base = wid * b_per_w
    nsteps = b_per_w

    pltpu.sync_copy(sel_hbm.at[pl.ds(base, b_per_w)], sel_v)

    lane = lax.iota(jnp.int32, 16)
    rep4 = lax.shift_right_logical(lane, 2)
    gdn = lax.GatherDimensionNumbers(
        offset_dims=(), collapsed_slice_dims=(0,), start_index_map=(0,))

    def start(t, q):
        b = base + t
        s16 = sel_v[t, pl.ds(q * 16, 16)]
        for c in range(4):
            sval = lax.gather(s16, (4 * c + rep4)[:, None], gdn, (1,),
                              mode=lax.GatherScatterMode.PROMISE_IN_BOUNDS)
            i16 = (q * _QROWS + c * 16) + lane
            idx_v[q, pl.ds(c * 16, 16)] = i16 * _NREP + sval
        pltpu.async_copy(
            lat_hbm.at[b, pl.ds(q * _QROWS, _QROWS)], lat_v.at[q], sem_l.at[q])
        pltpu.async_copy(tab_hbm.at[idx_v.at[q]], emb_v.at[q], sem_g.at[q])

    def finish(t, q):
        b = base + t
        pltpu.make_async_copy(
            lat_hbm.at[b, pl.ds(q * _QROWS, _QROWS)], lat_v.at[q],
            sem_l.at[q]).wait()
        pltpu.make_async_copy(
            tab_hbm.at[idx_v.at[q]], emb_v.at[q], sem_g.at[q]).wait()

        def addrow(r, inner):
            for rr in range(2):
                for ch in range(8):
                    sl = pl.ds(ch * 16, 16)
                    emb_v[q, 2 * r + rr, sl] = (
                        emb_v[q, 2 * r + rr, sl] + lat_v[q, 2 * r + rr, sl])
            return inner

        lax.fori_loop(0, _QROWS // 2, addrow, 0)
        pltpu.async_copy(
            emb_v.at[q], out_hbm.at[b, pl.ds(q * _QROWS, _QROWS)], sem_o.at[q])

    def drain_out(t, q):
        pltpu.make_async_copy(
            emb_v.at[q], out_hbm.at[base + t, pl.ds(q * _QROWS, _QROWS)],
            sem_o.at[q]).wait()

    start(0, 0)
    start(0, 1)
    start(0, 2)

    def step(t, carry):
        for p in range(4):
            finish(t, p)
            if p == 0:
                @pl.when(t >= 1)
                def _():
                    drain_out(t - 1, 3)
            else:
                drain_out(t, p - 1)
            if p == 0:
                start(t, 3)
            else:
                @pl.when(t < nsteps - 1)
                def _():
                    start(t + 1, p - 1)
        return carry

    lax.fori_loop(0, nsteps, step, 0)
    drain_out(nsteps - 1, 3)


def _sc_call(latents, sel, emb):
    b = latents.shape[0]
    tab = emb.reshape(_TAB_ROWS, _C)
    b_per_w = b // _NW
    run = functools.partial(
        pl.kernel,
        mesh=plsc.VectorSubcoreMesh(core_axis_name="c", subcore_axis_name="s"),
        out_type=jax.ShapeDtypeStruct((b, _E, _C), jnp.float32),
        scratch_types=[
            pltpu.VMEM((b_per_w, _S), jnp.int32),
            pltpu.VMEM((_NBUF, _QROWS), jnp.int32),
            pltpu.VMEM((_NBUF, _QROWS, _C), jnp.float32),
            pltpu.VMEM((_NBUF, _QROWS, _C), jnp.float32),
            pltpu.SemaphoreType.DMA((_NBUF,)),
            pltpu.SemaphoreType.DMA((_NBUF,)),
            pltpu.SemaphoreType.DMA((_NBUF,)),
        ],
        compiler_params=pltpu.CompilerParams(use_tc_tiling_on_sc=False),
    )(_sc_body)
    return run(latents, sel, tab)


def kernel(latents, selections, emb):
    sel = selections.astype(jnp.int32)
    return _sc_call(latents, sel, emb)


def _tc_kernel(latents, selections, emb):
    b = latents.shape[0]
    sel = selections.astype(jnp.int32)
    embt = (
        emb.reshape(_S, _M, _NREP, _C)
        .transpose(0, 2, 1, 3)
        .reshape(_S, _NREP, _ROW)
        .astype(jnp.bfloat16)
    )
    return pl.pallas_call(
        _body,
        grid=(b // _BBLK,),
        in_specs=[
            pl.BlockSpec((_BBLK, _S), lambda i: (i, 0)),
            pl.BlockSpec((_BBLK, _E, _C), lambda i: (i, 0, 0)),
            pl.BlockSpec((_S, _NREP, _ROW), lambda i: (0, 0, 0)),
        ],
        out_specs=pl.BlockSpec((_BBLK, _E, _C), lambda i: (i, 0, 0)),
        out_shape=jax.ShapeDtypeStruct((b, _E, _C), jnp.float32),
        compiler_params=pltpu.CompilerParams(
            dimension_semantics=("arbitrary",),
        ),
    )(sel, latents, embt)
